# Initial kernel scaffold; baseline (speedup 1.0000x reference)
import jax
import jax.numpy as jnp
from jax.experimental import pallas as pl


def kernel(tokens, embedding):
    raise NotImplementedError("write your pallas kernel here")


# speedup vs baseline: 1.3994x; 1.3994x over previous
import math

import jax
import jax.numpy as jnp
from jax import lax
from jax.experimental import pallas as pl
from jax.experimental.pallas import tpu as pltpu
from jax.experimental.pallas import tpu_sc as plsc

EMB = 32
SCALE = math.sqrt(EMB)

K = 128
CH = 8
C = K * CH


def _make_kernel(B, V):
    info = plsc.get_sparse_core_info()
    NW = info.num_cores * info.num_subcores
    rows_per_w = B // NW
    n_chunks = rows_per_w // C

    mesh = plsc.VectorSubcoreMesh(core_axis_name="c", subcore_axis_name="s")

    @pl.kernel(
        mesh=mesh,
        out_type=jax.ShapeDtypeStruct((B, EMB), jnp.float32),
        scratch_types=[
            pltpu.VMEM((CH, K), jnp.int32),
            pltpu.VMEM((C, EMB), jnp.float32),
            pltpu.SemaphoreType.DMA,
        ],
        compiler_params=pltpu.CompilerParams(use_tc_tiling_on_sc=False),
    )
    def k(tok_hbm, table_hbm, out_hbm, idx_v, rows_v, sem):
        wid = lax.axis_index("s") * info.num_cores + lax.axis_index("c")
        base = wid * rows_per_w

        def chunk_body(c, carry):
            row0 = pl.multiple_of(base + c * C, 8)
            pltpu.sync_copy(tok_hbm.at[pl.ds(pl.multiple_of(row0 // K, 8), CH)], idx_v)
            copies = [
                pltpu.async_copy(
                    table_hbm.at[idx_v.at[j]],
                    rows_v.at[pl.ds(j * K, K)],
                    sem,
                )
                for j in range(CH)
            ]
            for cp in copies:
                cp.wait()

            def scale_body(i, carry2):
                r = i * 8
                for u in range(8):
                    for h in range(EMB // 16):
                        sl = (r + u, pl.ds(h * 16, 16))
                        rows_v[sl] = rows_v[sl] * SCALE
                return carry2

            lax.fori_loop(0, C // 8, scale_body, 0)
            pltpu.sync_copy(rows_v, out_hbm.at[pl.ds(row0, C)])
            return carry

        lax.fori_loop(0, n_chunks, chunk_body, 0)

    return k


def kernel(tokens, embedding):
    B = tokens.shape[0] * tokens.shape[1]
    V = embedding.shape[0]
    tok2d = tokens.reshape(B // K, K).astype(jnp.int32)
    out = _make_kernel(B, V)(tok2d, embedding)
    return out.reshape(tokens.shape[0], tokens.shape[1], EMB)


# speedup vs baseline: 1.4771x; 1.0555x over previous
import math

import jax
import jax.numpy as jnp
from jax import lax
from jax.experimental import pallas as pl
from jax.experimental.pallas import tpu as pltpu
from jax.experimental.pallas import tpu_sc as plsc

EMB = 32
SCALE = math.sqrt(EMB)

K = 128
CH = 10
C = K * CH


def _make_kernel(B):
    info = plsc.get_sparse_core_info()
    NC = info.num_cores
    NW = NC * info.num_subcores
    RW = B // NW
    NCH = RW // C
    assert NCH % 2 == 0 and NCH * C == RW

    mesh = plsc.VectorSubcoreMesh(core_axis_name="c", subcore_axis_name="s")

    @pl.kernel(
        mesh=mesh,
        out_type=jax.ShapeDtypeStruct((B, EMB), jnp.float32),
        scratch_types=[
            pltpu.VMEM((RW // K, K), jnp.int32),
            pltpu.VMEM((2, C, EMB), jnp.float32),
            pltpu.SemaphoreType.DMA,
            pltpu.SemaphoreType.DMA,
            pltpu.SemaphoreType.DMA,
            pltpu.SemaphoreType.DMA,
        ],
        compiler_params=pltpu.CompilerParams(use_tc_tiling_on_sc=False),
    )
    def k(tok_hbm, table_hbm, out_hbm, idx_v, rows_v, sg0, sg1, so0, so1):
        wid = lax.axis_index("s") * NC + lax.axis_index("c")
        base = wid * RW
        semg = (sg0, sg1)
        semo = (so0, so1)

        pltpu.sync_copy(
            tok_hbm.at[pl.ds(pl.multiple_of(base // K, 8), RW // K)], idx_v)

        def fire(ci, b):
            for j in range(CH):
                pltpu.async_copy(
                    table_hbm.at[idx_v.at[ci * CH + j]],
                    rows_v.at[b, pl.ds(j * K, K)],
                    semg[b],
                )

        def wait_g(b):
            pltpu.make_async_copy(
                table_hbm.at[pl.ds(0, C)], rows_v.at[b], semg[b]).wait()

        def out_slice(ci):
            return out_hbm.at[pl.ds(pl.multiple_of(base + ci * C, 8), C)]

        def scale(b):
            def body(i, carry):
                r = i * 8
                for u in range(8):
                    for h in range(EMB // 16):
                        sl = (b, r + u, pl.ds(h * 16, 16))
                        rows_v[sl] = rows_v[sl] * SCALE
                return carry
            lax.fori_loop(0, C // 8, body, 0)

        fire(0, 0)

        def outer(c2, carry):
            for b in range(2):
                ci = c2 * 2 + b
                nb = 1 - b

                @pl.when(ci + 1 < NCH)
                def _fire_next():
                    @pl.when(ci >= 1)
                    def _drain_prev_out():
                        pltpu.make_async_copy(
                            rows_v.at[nb], out_slice(ci - 1), semo[nb]).wait()
                    fire(ci + 1, nb)

                wait_g(b)
                scale(b)
                pltpu.async_copy(rows_v.at[b], out_slice(ci), semo[b])
            return carry

        lax.fori_loop(0, NCH // 2, outer, 0)
        pltpu.make_async_copy(rows_v.at[0], out_slice(NCH - 2), semo[0]).wait()
        pltpu.make_async_copy(rows_v.at[1], out_slice(NCH - 1), semo[1]).wait()

    return k


def kernel(tokens, embedding):
    B = tokens.shape[0] * tokens.shape[1]
    tok2d = tokens.reshape(B // K, K).astype(jnp.int32)
    out = _make_kernel(B)(tok2d, embedding)
    return out.reshape(tokens.shape[0], tokens.shape[1], EMB)
